# baseline (speedup 1.0000x reference)
import functools

import jax
import jax.numpy as jnp
from jax import lax
from jax.experimental import pallas as pl
from jax.experimental.pallas import tpu as pltpu
from jax.experimental.pallas import tpu_sc as plsc

B = 16384
D = 128
L = 16
NC = 2
NS = 16
NW = NC * NS
B_PER_W = B // NW
CHUNK = 128
NCHUNK = B_PER_W // CHUNK
GROUPS = CHUNK // L


def _body(uidx_hbm, iidx_hbm, utab_hbm, itab_hbm, out_hbm,
          uidx_v, iidx_v, u_rows, i_rows, out_v, sem_u, sem_i):
  wid = lax.axis_index("s") * NC + lax.axis_index("c")
  base = wid * B_PER_W

  for c in range(NCHUNK):
    pltpu.sync_copy(uidx_hbm.at[pl.ds(base + c * CHUNK, CHUNK)], uidx_v.at[c])
    pltpu.sync_copy(iidx_hbm.at[pl.ds(base + c * CHUNK, CHUNK)], iidx_v.at[c])

  iot = lax.iota(jnp.int32, L)

  def chunk_body(c, carry):
    cu = pltpu.async_copy(utab_hbm.at[uidx_v.at[c]], u_rows, sem_u)
    ci = pltpu.async_copy(itab_hbm.at[iidx_v.at[c]], i_rows, sem_i)
    cu.wait()
    ci.wait()
    for g in range(GROUPS):
      res = jnp.zeros((L,), jnp.float32)
      for r in range(L):
        row = g * L + r
        acc = u_rows[row, 0:L] * i_rows[row, 0:L]
        for k in range(1, D // L):
          acc = acc + u_rows[row, k * L:(k + 1) * L] * i_rows[row, k * L:(k + 1) * L]
        res = jnp.where(iot == r, jnp.sum(acc), res)
      out_v[pl.ds(c * CHUNK + g * L, L)] = res
    return carry

  lax.fori_loop(0, NCHUNK, chunk_body, 0)
  pltpu.sync_copy(out_v, out_hbm.at[pl.ds(base, B_PER_W)])


@functools.partial(jax.jit, static_argnums=())
def _run(user_indices, item_indices, user_table, item_table):
  mesh = plsc.VectorSubcoreMesh(core_axis_name="c", subcore_axis_name="s")
  f = pl.kernel(
      _body,
      out_type=jax.ShapeDtypeStruct((B,), jnp.float32),
      mesh=mesh,
      compiler_params=pltpu.CompilerParams(needs_layout_passes=False),
      scratch_types=[
          pltpu.VMEM((NCHUNK, CHUNK), jnp.int32),
          pltpu.VMEM((NCHUNK, CHUNK), jnp.int32),
          pltpu.VMEM((CHUNK, D), jnp.float32),
          pltpu.VMEM((CHUNK, D), jnp.float32),
          pltpu.VMEM((B_PER_W,), jnp.float32),
          pltpu.SemaphoreType.DMA,
          pltpu.SemaphoreType.DMA,
      ],
  )
  return f(user_indices, item_indices, user_table, item_table)


def kernel(user_indices, item_indices, user_table, item_table):
  return _run(user_indices.astype(jnp.int32), item_indices.astype(jnp.int32),
              user_table, item_table)


# speedup vs baseline: 1.0765x; 1.0765x over previous
import functools

import jax
import jax.numpy as jnp
from jax import lax
from jax.experimental import pallas as pl
from jax.experimental.pallas import tpu as pltpu
from jax.experimental.pallas import tpu_sc as plsc

B = 16384
D = 128
L = 16
NC = 2
NS = 16
NW = NC * NS
B_PER_W = B // NW
CHUNK = 128
NCHUNK = B_PER_W // CHUNK
GROUPS = CHUNK // L


def _body(uidx_hbm, iidx_hbm, utab_hbm, itab_hbm, out_hbm,
          uidx_v, iidx_v, u_rows0, u_rows1, i_rows0, i_rows1, out_v,
          sem_u0, sem_u1, sem_i0, sem_i1):
  wid = lax.axis_index("s") * NC + lax.axis_index("c")
  base = wid * B_PER_W

  u_rows = [u_rows0, u_rows1]
  i_rows = [i_rows0, i_rows1]
  sem_u = [sem_u0, sem_u1]
  sem_i = [sem_i0, sem_i1]

  for c in range(NCHUNK):
    pltpu.sync_copy(uidx_hbm.at[pl.ds(base + c * CHUNK, CHUNK)], uidx_v.at[c])
    pltpu.sync_copy(iidx_hbm.at[pl.ds(base + c * CHUNK, CHUNK)], iidx_v.at[c])

  iot = lax.iota(jnp.int32, L)

  def start(c):
    p = c % 2
    cu = pltpu.async_copy(utab_hbm.at[uidx_v.at[c]], u_rows[p], sem_u[p])
    ci = pltpu.async_copy(itab_hbm.at[iidx_v.at[c]], i_rows[p], sem_i[p])
    return cu, ci

  pend = start(0)
  for c in range(NCHUNK):
    p = c % 2
    pend[0].wait()
    pend[1].wait()
    if c + 1 < NCHUNK:
      pend = start(c + 1)
    ur, ir = u_rows[p], i_rows[p]

    def group_body(g, carry, ur=ur, ir=ir, c=c):
      res = jnp.zeros((L,), jnp.float32)
      for r in range(L):
        row = g * L + r
        acc = ur[row, 0:L] * ir[row, 0:L]
        for k in range(1, D // L):
          acc = acc + ur[row, k * L:(k + 1) * L] * ir[row, k * L:(k + 1) * L]
        res = jnp.where(iot == r, jnp.sum(acc), res)
      out_v[pl.ds(c * CHUNK + g * L, L)] = res
      return carry

    lax.fori_loop(0, GROUPS, group_body, 0)

  pltpu.sync_copy(out_v, out_hbm.at[pl.ds(base, B_PER_W)])


@functools.partial(jax.jit, static_argnums=())
def _run(user_indices, item_indices, user_table, item_table):
  mesh = plsc.VectorSubcoreMesh(core_axis_name="c", subcore_axis_name="s")
  f = pl.kernel(
      _body,
      out_type=jax.ShapeDtypeStruct((B,), jnp.float32),
      mesh=mesh,
      compiler_params=pltpu.CompilerParams(needs_layout_passes=False),
      scratch_types=[
          pltpu.VMEM((NCHUNK, CHUNK), jnp.int32),
          pltpu.VMEM((NCHUNK, CHUNK), jnp.int32),
          pltpu.VMEM((CHUNK, D), jnp.float32),
          pltpu.VMEM((CHUNK, D), jnp.float32),
          pltpu.VMEM((CHUNK, D), jnp.float32),
          pltpu.VMEM((CHUNK, D), jnp.float32),
          pltpu.VMEM((B_PER_W,), jnp.float32),
          pltpu.SemaphoreType.DMA,
          pltpu.SemaphoreType.DMA,
          pltpu.SemaphoreType.DMA,
          pltpu.SemaphoreType.DMA,
      ],
  )
  return f(user_indices, item_indices, user_table, item_table)


def kernel(user_indices, item_indices, user_table, item_table):
  return _run(user_indices.astype(jnp.int32), item_indices.astype(jnp.int32),
              user_table, item_table)


# speedup vs baseline: 1.4961x; 1.3898x over previous
import functools

import jax
import jax.numpy as jnp
from jax import lax
from jax.experimental import pallas as pl
from jax.experimental.pallas import tpu as pltpu
from jax.experimental.pallas import tpu_sc as plsc

B = 16384
D = 128
L = 16
NC = 2
NS = 16
NW = NC * NS
B_PER_W = B // NW
CHUNK = 128
NCHUNK = B_PER_W // CHUNK
GROUPS = CHUNK // L


def _body(uidx_hbm, iidx_hbm, utab_hbm, itab_hbm, out_hbm,
          uidx_v, iidx_v, u_rows0, u_rows1, i_rows0, i_rows1, part, out_v,
          sem_u0, sem_u1, sem_i0, sem_i1):
  wid = lax.axis_index("s") * NC + lax.axis_index("c")
  base = wid * B_PER_W

  u_rows = [u_rows0, u_rows1]
  i_rows = [i_rows0, i_rows1]
  sem_u = [sem_u0, sem_u1]
  sem_i = [sem_i0, sem_i1]

  for c in range(NCHUNK):
    pltpu.sync_copy(uidx_hbm.at[pl.ds(base + c * CHUNK, CHUNK)], uidx_v.at[c])
    pltpu.sync_copy(iidx_hbm.at[pl.ds(base + c * CHUNK, CHUNK)], iidx_v.at[c])

  iot = lax.iota(jnp.int32, L)

  def start(c):
    p = c % 2
    cu = pltpu.async_copy(utab_hbm.at[uidx_v.at[c]], u_rows[p], sem_u[p])
    ci = pltpu.async_copy(itab_hbm.at[iidx_v.at[c]], i_rows[p], sem_i[p])
    return cu, ci

  pend = start(0)
  for c in range(NCHUNK):
    p = c % 2
    pend[0].wait()
    pend[1].wait()
    if c + 1 < NCHUNK:
      pend = start(c + 1)
    ur, ir = u_rows[p], i_rows[p]

    def group_body(g, carry, ur=ur, ir=ir, c=c):
      for r in range(L):
        row = g * L + r
        acc = ur[row, 0:L] * ir[row, 0:L]
        for k in range(1, D // L):
          acc = acc + ur[row, k * L:(k + 1) * L] * ir[row, k * L:(k + 1) * L]
        part[pl.ds(r * (L + 1), L)] = acc
      res = plsc.load_gather(part, [iot * (L + 1)])
      for cc in range(1, L):
        res = res + plsc.load_gather(part, [iot * (L + 1) + cc])
      out_v[pl.ds(c * CHUNK + g * L, L)] = res
      return carry

    lax.fori_loop(0, GROUPS, group_body, 0)

  pltpu.sync_copy(out_v, out_hbm.at[pl.ds(base, B_PER_W)])


@functools.partial(jax.jit, static_argnums=())
def _run(user_indices, item_indices, user_table, item_table):
  mesh = plsc.VectorSubcoreMesh(core_axis_name="c", subcore_axis_name="s")
  f = pl.kernel(
      _body,
      out_type=jax.ShapeDtypeStruct((B,), jnp.float32),
      mesh=mesh,
      compiler_params=pltpu.CompilerParams(needs_layout_passes=False),
      scratch_types=[
          pltpu.VMEM((NCHUNK, CHUNK), jnp.int32),
          pltpu.VMEM((NCHUNK, CHUNK), jnp.int32),
          pltpu.VMEM((CHUNK, D), jnp.float32),
          pltpu.VMEM((CHUNK, D), jnp.float32),
          pltpu.VMEM((CHUNK, D), jnp.float32),
          pltpu.VMEM((CHUNK, D), jnp.float32),
          pltpu.VMEM((L * (L + 1),), jnp.float32),
          pltpu.VMEM((B_PER_W,), jnp.float32),
          pltpu.SemaphoreType.DMA,
          pltpu.SemaphoreType.DMA,
          pltpu.SemaphoreType.DMA,
          pltpu.SemaphoreType.DMA,
      ],
  )
  return f(user_indices, item_indices, user_table, item_table)


def kernel(user_indices, item_indices, user_table, item_table):
  return _run(user_indices.astype(jnp.int32), item_indices.astype(jnp.int32),
              user_table, item_table)


# speedup vs baseline: 1.5421x; 1.0307x over previous
import functools

import jax
import jax.numpy as jnp
from jax import lax
from jax.experimental import pallas as pl
from jax.experimental.pallas import tpu as pltpu
from jax.experimental.pallas import tpu_sc as plsc

B = 16384
D = 128
L = 16
NC = 2
NS = 16
NW = NC * NS
B_PER_W = B // NW
CHUNK = 128
NCHUNK = B_PER_W // CHUNK
GROUPS = CHUNK // L
NBUF = 3


def _body(uidx_hbm, iidx_hbm, utab_hbm, itab_hbm, out_hbm,
          uidx_v, iidx_v, u0, u1, u2, i0, i1, i2, part, out_v,
          su0, su1, su2, si0, si1, si2, sx):
  wid = lax.axis_index("s") * NC + lax.axis_index("c")
  base = wid * B_PER_W

  ubufs = [u0, u1, u2]
  ibufs = [i0, i1, i2]
  sus = [su0, su1, su2]
  sis = [si0, si1, si2]

  stage = []
  for c in range(NCHUNK):
    stage.append(pltpu.async_copy(
        uidx_hbm.at[pl.ds(base + c * CHUNK, CHUNK)], uidx_v.at[c], sx))
    stage.append(pltpu.async_copy(
        iidx_hbm.at[pl.ds(base + c * CHUNK, CHUNK)], iidx_v.at[c], sx))
  for cp in stage:
    cp.wait()

  iot = lax.iota(jnp.int32, L)

  def start(c):
    p = c % NBUF
    cu = pltpu.async_copy(utab_hbm.at[uidx_v.at[c]], ubufs[p], sus[p])
    ci = pltpu.async_copy(itab_hbm.at[iidx_v.at[c]], ibufs[p], sis[p])
    return cu, ci

  pend = {}
  for c in range(min(NBUF, NCHUNK)):
    pend[c] = start(c)

  for c in range(NCHUNK):
    p = c % NBUF
    pend[c][0].wait()
    pend[c][1].wait()
    ur, ir = ubufs[p], ibufs[p]

    def group_body(g, carry, ur=ur, ir=ir, c=c):
      for r in range(L):
        row = g * L + r
        acc = ur[row, 0:L] * ir[row, 0:L]
        for k in range(1, D // L):
          acc = acc + ur[row, k * L:(k + 1) * L] * ir[row, k * L:(k + 1) * L]
        part[pl.ds(r * (L + 1), L)] = acc
      res = plsc.load_gather(part, [iot * (L + 1)])
      for cc in range(1, L):
        res = res + plsc.load_gather(part, [iot * (L + 1) + cc])
      out_v[pl.ds(c * CHUNK + g * L, L)] = res
      return carry

    lax.fori_loop(0, GROUPS, group_body, 0)
    if c + NBUF < NCHUNK:
      pend[c + NBUF] = start(c + NBUF)

  pltpu.sync_copy(out_v, out_hbm.at[pl.ds(base, B_PER_W)])


@jax.jit
def _run(user_indices, item_indices, user_table, item_table):
  mesh = plsc.VectorSubcoreMesh(core_axis_name="c", subcore_axis_name="s")
  f = pl.kernel(
      _body,
      out_type=jax.ShapeDtypeStruct((B,), jnp.float32),
      mesh=mesh,
      compiler_params=pltpu.CompilerParams(needs_layout_passes=False),
      scratch_types=[
          pltpu.VMEM((NCHUNK, CHUNK), jnp.int32),
          pltpu.VMEM((NCHUNK, CHUNK), jnp.int32),
          pltpu.VMEM((CHUNK, D), jnp.float32),
          pltpu.VMEM((CHUNK, D), jnp.float32),
          pltpu.VMEM((CHUNK, D), jnp.float32),
          pltpu.VMEM((CHUNK, D), jnp.float32),
          pltpu.VMEM((CHUNK, D), jnp.float32),
          pltpu.VMEM((CHUNK, D), jnp.float32),
          pltpu.VMEM((L * (L + 1),), jnp.float32),
          pltpu.VMEM((B_PER_W,), jnp.float32),
          pltpu.SemaphoreType.DMA,
          pltpu.SemaphoreType.DMA,
          pltpu.SemaphoreType.DMA,
          pltpu.SemaphoreType.DMA,
          pltpu.SemaphoreType.DMA,
          pltpu.SemaphoreType.DMA,
          pltpu.SemaphoreType.DMA,
      ],
  )
  return f(user_indices, item_indices, user_table, item_table)


def kernel(user_indices, item_indices, user_table, item_table):
  return _run(user_indices.astype(jnp.int32), item_indices.astype(jnp.int32),
              user_table, item_table)
